# baseline (speedup 1.0000x reference)
import functools

import jax
import jax.numpy as jnp
from jax import lax
from jax.experimental import pallas as pl
from jax.experimental.pallas import tpu as pltpu
from jax.experimental.pallas import tpu_sc as plsc

N = 10000
E = 160000
IN_DIM = 256
HID = 512
OUT_DIM = 256
NLAYERS = 4

NCORES = 2
NSUB = 16
NW = NCORES * NSUB
EC = E // NSUB
NVREG = (EC + 15) // 16
RP = 1280
NPART = 8
NPAD = RP * NPART
DUMMY = RP
ACC_ROWS = RP + 8 * NSUB
ZROWS = 16
G = 64
WCAP = 10368


def _make_agg(D):
    FACT = D // 128
    SH = FACT.bit_length() - 1
    GO = 128 // FACT
    NPART_D = NPART // (4 // FACT) if FACT < 4 else NPART
    RP_D = NPAD // NPART_D
    PASSES = NPART_D // NCORES
    ACC_D = RP_D + 8 * NSUB
    AF = ACC_D * FACT
    ZT = AF // NSUB
    zrounds = ZT // ZROWS
    zrem = ZT - zrounds * ZROWS

    @functools.partial(
        pl.kernel,
        mesh=plsc.VectorSubcoreMesh(core_axis_name="c", subcore_axis_name="s"),
        compiler_params=pltpu.CompilerParams(needs_layout_passes=False),
        out_type=jax.ShapeDtypeStruct((NPAD * FACT, 128), jnp.float32),
        scratch_types=[
            pltpu.VMEM_SHARED((AF, 128), jnp.float32),
            pltpu.VMEM((NVREG * 16,), jnp.int32),
            pltpu.VMEM((NVREG * 16,), jnp.int32),
            pltpu.VMEM((WCAP,), jnp.int32),
            pltpu.VMEM((WCAP,), jnp.int32),
            pltpu.VMEM((128,), jnp.int32),
            pltpu.VMEM((128,), jnp.int32),
            pltpu.VMEM((128, 128), jnp.float32),
            pltpu.VMEM((ZROWS, 128), jnp.float32),
            pltpu.SemaphoreType.DMA,
        ],
    )
    def agg(table, srce, dste, out, acc, srcb, dstb, wsrc, wdst, wgix, wdix,
            rows, zbuf, sem):
        cid = lax.axis_index("c")
        sid = lax.axis_index("s")
        z16f = jnp.zeros((16,), jnp.float32)
        z16i = jnp.zeros((16,), jnp.int32)
        dummy16 = jnp.full((16,), RP_D, jnp.int32)
        lane = lax.iota(jnp.int32, 16)

        def zb_body(i, _):
            zbuf[i // 8, pl.ds((i % 8) * 16, 16)] = z16f
            return 0

        lax.fori_loop(0, ZROWS * 8, zb_body, 0)

        pltpu.sync_copy(srce.at[pl.ds(sid * EC, EC)], srcb.at[pl.ds(0, EC)])
        pltpu.sync_copy(dste.at[pl.ds(sid * EC, EC)], dstb.at[pl.ds(0, EC)])
        tbase = (NVREG - 1) * 16
        tv = dstb[pl.ds(tbase, 16)]
        dstb[pl.ds(tbase, 16)] = jnp.where(lane < (EC - tbase), tv, NPAD)

        def do_pass(p, _):
            lo = (cid * PASSES + p) * RP_D
            zb = sid * ZT
            for q in range(zrounds):
                pltpu.sync_copy(zbuf, acc.at[pl.ds(zb + q * ZROWS, ZROWS)])
            if zrem:
                pltpu.sync_copy(zbuf.at[pl.ds(0, zrem)],
                                acc.at[pl.ds(zb + zrounds * ZROWS, zrem)])

            def pre(i, _):
                wsrc[pl.ds(i * 16, 16)] = z16i
                wdst[pl.ds(i * 16, 16)] = dummy16
                return 0

            lax.fori_loop(0, WCAP // 16, pre, 0)

            def comp(i, cnt):
                dv = dstb[pl.ds(i * 16, 16)]
                sv = srcb[pl.ds(i * 16, 16)]
                m = (dv >= lo) & (dv < lo + RP_D)
                mi = m.astype(jnp.int32)
                pos = plsc.cumsum(mi) - 1 + cnt
                plsc.store_scatter(wsrc, [pos], sv, mask=m)
                plsc.store_scatter(wdst, [pos], dv - lo, mask=m)
                return cnt + jnp.sum(mi)

            cnt = lax.fori_loop(0, NVREG, comp, 0)
            plsc.subcore_barrier()

            nch = (cnt + GO - 1) // GO

            def chunk(j, _):
                jb = j * GO
                for t in range(8):
                    if FACT == 1:
                        gsl = jb + t * 16 + lane
                        sub = 0
                    else:
                        gsl = jb + (t * 16) // FACT + (lane >> SH)
                        sub = lane & (FACT - 1)
                    sv = plsc.load_gather(wsrc, [gsl])
                    dv = plsc.load_gather(wdst, [gsl])
                    wgix[pl.ds(t * 16, 16)] = sv * FACT + sub
                    wdix[pl.ds(t * 16, 16)] = dv * FACT + sub
                pltpu.async_copy(table.at[wgix], rows, sem).wait()
                pltpu.sync_copy(rows, acc.at[wdix], add=True)
                return 0

            lax.fori_loop(0, nch, chunk, 0)
            plsc.subcore_barrier()

            ot = (RP_D * FACT) // NSUB
            ob = sid * ot
            pltpu.sync_copy(acc.at[pl.ds(ob, ot)],
                            out.at[pl.ds(lo * FACT + ob, ot)])
            plsc.subcore_barrier()
            return 0

        lax.fori_loop(0, PASSES, do_pass, 0)

    return agg


_agg_wide = _make_agg(HID)
_agg_deg = _make_agg(128)

BR = 1000


def _in_body(x_ref, w_ref, b_ref, o_ref):
    o_ref[...] = (
        jnp.dot(x_ref[...], w_ref[...], preferred_element_type=jnp.float32)
        + b_ref[...])


def _mm_body(h_ref, wc_ref, wr_ref, br_ref, d0_ref, xs_ref, res_ref):
    h = h_ref[...]
    deg = d0_ref[:, 0:1]
    dinv = lax.rsqrt(deg + 1.0)
    xs_ref[...] = (
        jnp.dot(h, wc_ref[...], preferred_element_type=jnp.float32) * dinv)
    res_ref[...] = (
        jnp.dot(h, wr_ref[...], preferred_element_type=jnp.float32)
        + br_ref[...])


def _post_body(a0_ref, a1_ref, xs_ref, res_ref, d0_ref, d1_ref, bc_ref, g_ref,
               be_ref, o_ref):
    deg = d0_ref[0, :, 0:1] + d1_ref[0, :, 0:1]
    dinv = lax.rsqrt(deg + 1.0)
    s = ((a0_ref[0] + a1_ref[0] + xs_ref[...]) * dinv + bc_ref[...]
         + res_ref[...])
    mu = jnp.mean(s, axis=1, keepdims=True)
    c = s - mu
    var = jnp.mean(c * c, axis=1, keepdims=True)
    y = c * lax.rsqrt(var + 1e-5) * g_ref[...] + be_ref[...]
    o_ref[...] = jnp.maximum(y, 0.0)


def _post_mm_body(a0_ref, xs_ref, res_ref, d0_ref, bc_ref,
                  g_ref, be_ref, wc_ref, wr_ref, br_ref, nxs_ref, nres_ref):
    deg = d0_ref[:, 0:1]
    dinv = lax.rsqrt(deg + 1.0)
    s = ((a0_ref[...] + xs_ref[...]) * dinv + bc_ref[...]
         + res_ref[...])
    mu = jnp.mean(s, axis=1, keepdims=True)
    c = s - mu
    var = jnp.mean(c * c, axis=1, keepdims=True)
    y = c * lax.rsqrt(var + 1e-5) * g_ref[...] + be_ref[...]
    h = jnp.maximum(y, 0.0)
    nxs_ref[...] = (
        jnp.dot(h, wc_ref[...], preferred_element_type=jnp.float32) * dinv)
    nres_ref[...] = (
        jnp.dot(h, wr_ref[...], preferred_element_type=jnp.float32)
        + br_ref[...])


def _post_out_body(a0_ref, xs_ref, res_ref, d0_ref, bc_ref,
                   g_ref, be_ref, wo_ref, bo_ref, o_ref):
    deg = d0_ref[:, 0:1]
    dinv = lax.rsqrt(deg + 1.0)
    s = ((a0_ref[...] + xs_ref[...]) * dinv + bc_ref[...]
         + res_ref[...])
    mu = jnp.mean(s, axis=1, keepdims=True)
    c = s - mu
    var = jnp.mean(c * c, axis=1, keepdims=True)
    y = c * lax.rsqrt(var + 1e-5) * g_ref[...] + be_ref[...]
    h = jnp.maximum(y, 0.0)
    o_ref[...] = (
        jnp.dot(h, wo_ref[...], preferred_element_type=jnp.float32)
        + bo_ref[...])


def _out_body(h_ref, w_ref, b_ref, o_ref):
    o_ref[...] = (
        jnp.dot(h_ref[...], w_ref[...], preferred_element_type=jnp.float32)
        + b_ref[...])


def _row_spec(d):
    return pl.BlockSpec((BR, d), lambda i: (i, 0))


def _full_spec(r, c):
    return pl.BlockSpec((r, c), lambda i: (0, 0))


def _part_spec(part, d):
    return pl.BlockSpec((1, BR, d), lambda i, part=part: (part, i, 0))


def kernel(x, edge_index, params):
    src = edge_index[0].astype(jnp.int32)
    dst = edge_index[1].astype(jnp.int32)
    p = params

    ones_tab = jnp.ones((N, 128), jnp.float32)
    deg2 = _agg_deg(ones_tab, src, dst)

    h = pl.pallas_call(
        _in_body,
        grid=(N // BR,),
        in_specs=[_row_spec(IN_DIM), _full_spec(IN_DIM, HID),
                  _full_spec(1, HID)],
        out_specs=_row_spec(HID),
        out_shape=jax.ShapeDtypeStruct((N, HID), jnp.float32),
    )(x, p['W_in'], p['b_in'].reshape(1, HID))

    xs, res = pl.pallas_call(
        _mm_body,
        grid=(N // BR,),
        in_specs=[_row_spec(HID), _full_spec(HID, HID),
                  _full_spec(HID, HID), _full_spec(1, HID),
                  _row_spec(128)],
        out_specs=[_row_spec(HID), _row_spec(HID)],
        out_shape=[jax.ShapeDtypeStruct((N, HID), jnp.float32),
                   jax.ShapeDtypeStruct((N, HID), jnp.float32)],
    )(h, p['Wc0'], p['Wr0'], p['br0'].reshape(1, HID), deg2)

    for l in range(NLAYERS - 1):
        agg2 = _agg_wide(xs.reshape(N * 4, 128), src, dst)
        agg2 = agg2.reshape(NPAD, HID)
        xs, res = pl.pallas_call(
            _post_mm_body,
            grid=(N // BR,),
            in_specs=[_row_spec(HID), _row_spec(HID),
                      _row_spec(HID), _row_spec(128),
                      _full_spec(1, HID), _full_spec(1, HID),
                      _full_spec(1, HID), _full_spec(HID, HID),
                      _full_spec(HID, HID), _full_spec(1, HID)],
            out_specs=[_row_spec(HID), _row_spec(HID)],
            out_shape=[jax.ShapeDtypeStruct((N, HID), jnp.float32),
                       jax.ShapeDtypeStruct((N, HID), jnp.float32)],
        )(agg2, xs, res, deg2, p[f'bc{l}'].reshape(1, HID),
          p[f'g{l}'].reshape(1, HID), p[f'be{l}'].reshape(1, HID),
          p[f'Wc{l + 1}'], p[f'Wr{l + 1}'], p[f'br{l + 1}'].reshape(1, HID))

    agg2 = _agg_wide(xs.reshape(N * 4, 128), src, dst)
    agg2 = agg2.reshape(NPAD, HID)
    l = NLAYERS - 1
    return pl.pallas_call(
        _post_out_body,
        grid=(N // BR,),
        in_specs=[_row_spec(HID), _row_spec(HID),
                  _row_spec(HID), _row_spec(128),
                  _full_spec(1, HID), _full_spec(1, HID), _full_spec(1, HID),
                  _full_spec(HID, OUT_DIM), _full_spec(1, OUT_DIM)],
        out_specs=_row_spec(OUT_DIM),
        out_shape=jax.ShapeDtypeStruct((N, OUT_DIM), jnp.float32),
    )(agg2, xs, res, deg2, p[f'bc{l}'].reshape(1, HID),
      p[f'g{l}'].reshape(1, HID), p[f'be{l}'].reshape(1, HID),
      p['W_out'], p['b_out'].reshape(1, OUT_DIM))


# speedup vs baseline: 1.0018x; 1.0018x over previous
import functools

import jax
import jax.numpy as jnp
from jax import lax
from jax.experimental import pallas as pl
from jax.experimental.pallas import tpu as pltpu
from jax.experimental.pallas import tpu_sc as plsc

N = 10000
E = 160000
IN_DIM = 256
HID = 512
OUT_DIM = 256
NLAYERS = 4

NCORES = 2
NSUB = 16
EC = E // NSUB
NVREG = (EC + 15) // 16
RP = 1280
NPART = 8
NPAD = RP * NPART
ACC_ROWS = RP + 8 * NSUB
ZROWS = 16
G = 64
WCAP = 10368


def _make_agg(D):
    FACT = D // 128
    SH = FACT.bit_length() - 1
    GO = 128 // FACT
    NPART_D = NPART // (4 // FACT) if FACT < 4 else NPART
    RP_D = NPAD // NPART_D
    PASSES = NPART_D // NCORES
    ACC_D = RP_D + 8 * NSUB
    AF = ACC_D * FACT
    ZT = AF // NSUB
    zrounds = ZT // ZROWS
    zrem = ZT - zrounds * ZROWS

    @functools.partial(
        pl.kernel,
        mesh=plsc.VectorSubcoreMesh(core_axis_name="c", subcore_axis_name="s"),
        compiler_params=pltpu.CompilerParams(needs_layout_passes=False),
        out_type=jax.ShapeDtypeStruct((NPAD * FACT, 128), jnp.float32),
        scratch_types=[
            pltpu.VMEM_SHARED((AF, 128), jnp.float32),
            pltpu.VMEM((NVREG * 16,), jnp.int32),
            pltpu.VMEM((NVREG * 16,), jnp.int32),
            pltpu.VMEM((WCAP,), jnp.int32),
            pltpu.VMEM((WCAP,), jnp.int32),
            pltpu.VMEM((128,), jnp.int32),
            pltpu.VMEM((128,), jnp.int32),
            pltpu.VMEM((128, 128), jnp.float32),
            pltpu.VMEM((ZROWS, 128), jnp.float32),
            pltpu.SemaphoreType.DMA,
        ],
    )
    def agg(table, srce, dste, out, acc, srcb, dstb, wsrc, wdst, wgix, wdix,
            rows, zbuf, sem):
        cid = lax.axis_index("c")
        sid = lax.axis_index("s")
        z16f = jnp.zeros((16,), jnp.float32)
        z16i = jnp.zeros((16,), jnp.int32)
        dummy16 = jnp.full((16,), RP_D, jnp.int32)
        lane = lax.iota(jnp.int32, 16)

        def zb_body(i, _):
            zbuf[i // 8, pl.ds((i % 8) * 16, 16)] = z16f
            return 0

        lax.fori_loop(0, ZROWS * 8, zb_body, 0)

        pltpu.sync_copy(srce.at[pl.ds(sid * EC, EC)], srcb.at[pl.ds(0, EC)])
        pltpu.sync_copy(dste.at[pl.ds(sid * EC, EC)], dstb.at[pl.ds(0, EC)])
        tbase = (NVREG - 1) * 16
        tv = dstb[pl.ds(tbase, 16)]
        dstb[pl.ds(tbase, 16)] = jnp.where(lane < (EC - tbase), tv, NPAD)

        def do_pass(p, _):
            lo = (cid * PASSES + p) * RP_D
            zb = sid * ZT
            for q in range(zrounds):
                pltpu.sync_copy(zbuf, acc.at[pl.ds(zb + q * ZROWS, ZROWS)])
            if zrem:
                pltpu.sync_copy(zbuf.at[pl.ds(0, zrem)],
                                acc.at[pl.ds(zb + zrounds * ZROWS, zrem)])

            def pre(i, _):
                wsrc[pl.ds(i * 16, 16)] = z16i
                wdst[pl.ds(i * 16, 16)] = dummy16
                return 0

            lax.fori_loop(0, WCAP // 16, pre, 0)

            def comp(i, cnt):
                dv = dstb[pl.ds(i * 16, 16)]
                sv = srcb[pl.ds(i * 16, 16)]
                m = (dv >= lo) & (dv < lo + RP_D)
                mi = m.astype(jnp.int32)
                pos = plsc.cumsum(mi) - 1 + cnt
                plsc.store_scatter(wsrc, [pos], sv, mask=m)
                plsc.store_scatter(wdst, [pos], dv - lo, mask=m)
                return cnt + jnp.sum(mi)

            cnt = lax.fori_loop(0, NVREG, comp, 0)
            plsc.subcore_barrier()

            nch = (cnt + GO - 1) // GO

            def chunk(j, _):
                jb = j * GO
                for t in range(8):
                    if FACT == 1:
                        gsl = jb + t * 16 + lane
                        sub = 0
                    else:
                        gsl = jb + (t * 16) // FACT + (lane >> SH)
                        sub = lane & (FACT - 1)
                    sv = plsc.load_gather(wsrc, [gsl])
                    dv = plsc.load_gather(wdst, [gsl])
                    wgix[pl.ds(t * 16, 16)] = sv * FACT + sub
                    wdix[pl.ds(t * 16, 16)] = dv * FACT + sub
                pltpu.async_copy(table.at[wgix], rows, sem).wait()
                pltpu.sync_copy(rows, acc.at[wdix], add=True)
                return 0

            lax.fori_loop(0, nch, chunk, 0)
            plsc.subcore_barrier()

            ot = (RP_D * FACT) // NSUB
            ob = sid * ot
            pltpu.sync_copy(acc.at[pl.ds(ob, ot)],
                            out.at[pl.ds(lo * FACT + ob, ot)])
            plsc.subcore_barrier()
            return 0

        lax.fori_loop(0, PASSES, do_pass, 0)

    return agg


_agg_wide = _make_agg(HID)
_agg_deg = _make_agg(128)

BR = 1000


def _in_body(x_ref, w_ref, b_ref, o_ref):
    o_ref[...] = (
        jnp.dot(x_ref[...], w_ref[...], preferred_element_type=jnp.float32)
        + b_ref[...])


def _mm_body(h_ref, wc_ref, wr_ref, br_ref, d0_ref, xs_ref, res_ref):
    h = h_ref[...]
    deg = d0_ref[:, 0:1]
    dinv = lax.rsqrt(deg + 1.0)
    xs_ref[...] = (
        jnp.dot(h, wc_ref[...], preferred_element_type=jnp.float32) * dinv)
    res_ref[...] = (
        jnp.dot(h, wr_ref[...], preferred_element_type=jnp.float32)
        + br_ref[...])


def _post_mm_body(a0_ref, xs_ref, res_ref, d0_ref, bc_ref,
                  g_ref, be_ref, wc_ref, wr_ref, br_ref, nxs_ref, nres_ref):
    deg = d0_ref[:, 0:1]
    dinv = lax.rsqrt(deg + 1.0)
    s = ((a0_ref[...] + xs_ref[...]) * dinv + bc_ref[...]
         + res_ref[...])
    mu = jnp.mean(s, axis=1, keepdims=True)
    c = s - mu
    var = jnp.mean(c * c, axis=1, keepdims=True)
    y = c * lax.rsqrt(var + 1e-5) * g_ref[...] + be_ref[...]
    h = jnp.maximum(y, 0.0)
    nxs_ref[...] = (
        jnp.dot(h, wc_ref[...], preferred_element_type=jnp.float32) * dinv)
    nres_ref[...] = (
        jnp.dot(h, wr_ref[...], preferred_element_type=jnp.float32)
        + br_ref[...])


def _post_out_body(a0_ref, xs_ref, res_ref, d0_ref, bc_ref,
                   g_ref, be_ref, wo_ref, bo_ref, o_ref):
    deg = d0_ref[:, 0:1]
    dinv = lax.rsqrt(deg + 1.0)
    s = ((a0_ref[...] + xs_ref[...]) * dinv + bc_ref[...]
         + res_ref[...])
    mu = jnp.mean(s, axis=1, keepdims=True)
    c = s - mu
    var = jnp.mean(c * c, axis=1, keepdims=True)
    y = c * lax.rsqrt(var + 1e-5) * g_ref[...] + be_ref[...]
    h = jnp.maximum(y, 0.0)
    o_ref[...] = (
        jnp.dot(h, wo_ref[...], preferred_element_type=jnp.float32)
        + bo_ref[...])


def _row_spec(d):
    return pl.BlockSpec((BR, d), lambda i: (i, 0))


def _full_spec(r, c):
    return pl.BlockSpec((r, c), lambda i: (0, 0))


def kernel(x, edge_index, params):
    src = edge_index[0].astype(jnp.int32)
    dst = edge_index[1].astype(jnp.int32)
    p = params

    ones_tab = jnp.ones((N, 128), jnp.float32)
    deg2 = _agg_deg(ones_tab, src, dst)

    h = pl.pallas_call(
        _in_body,
        grid=(N // BR,),
        in_specs=[_row_spec(IN_DIM), _full_spec(IN_DIM, HID),
                  _full_spec(1, HID)],
        out_specs=_row_spec(HID),
        out_shape=jax.ShapeDtypeStruct((N, HID), jnp.float32),
    )(x, p['W_in'], p['b_in'].reshape(1, HID))

    xs, res = pl.pallas_call(
        _mm_body,
        grid=(N // BR,),
        in_specs=[_row_spec(HID), _full_spec(HID, HID),
                  _full_spec(HID, HID), _full_spec(1, HID),
                  _row_spec(128)],
        out_specs=[_row_spec(HID), _row_spec(HID)],
        out_shape=[jax.ShapeDtypeStruct((N, HID), jnp.float32),
                   jax.ShapeDtypeStruct((N, HID), jnp.float32)],
    )(h, p['Wc0'], p['Wr0'], p['br0'].reshape(1, HID), deg2)

    for l in range(NLAYERS - 1):
        agg2 = _agg_wide(xs.reshape(N * 4, 128), src, dst)
        agg2 = agg2.reshape(NPAD, HID)
        xs, res = pl.pallas_call(
            _post_mm_body,
            grid=(N // BR,),
            in_specs=[_row_spec(HID), _row_spec(HID),
                      _row_spec(HID), _row_spec(128),
                      _full_spec(1, HID), _full_spec(1, HID),
                      _full_spec(1, HID), _full_spec(HID, HID),
                      _full_spec(HID, HID), _full_spec(1, HID)],
            out_specs=[_row_spec(HID), _row_spec(HID)],
            out_shape=[jax.ShapeDtypeStruct((N, HID), jnp.float32),
                       jax.ShapeDtypeStruct((N, HID), jnp.float32)],
        )(agg2, xs, res, deg2, p[f'bc{l}'].reshape(1, HID),
          p[f'g{l}'].reshape(1, HID), p[f'be{l}'].reshape(1, HID),
          p[f'Wc{l + 1}'], p[f'Wr{l + 1}'], p[f'br{l + 1}'].reshape(1, HID))

    agg2 = _agg_wide(xs.reshape(N * 4, 128), src, dst)
    agg2 = agg2.reshape(NPAD, HID)
    l = NLAYERS - 1
    return pl.pallas_call(
        _post_out_body,
        grid=(N // BR,),
        in_specs=[_row_spec(HID), _row_spec(HID),
                  _row_spec(HID), _row_spec(128),
                  _full_spec(1, HID), _full_spec(1, HID), _full_spec(1, HID),
                  _full_spec(HID, OUT_DIM), _full_spec(1, OUT_DIM)],
        out_specs=_row_spec(OUT_DIM),
        out_shape=jax.ShapeDtypeStruct((N, OUT_DIM), jnp.float32),
    )(agg2, xs, res, deg2, p[f'bc{l}'].reshape(1, HID),
      p[f'g{l}'].reshape(1, HID), p[f'be{l}'].reshape(1, HID),
      p['W_out'], p['b_out'].reshape(1, OUT_DIM))
